# baseline (speedup 1.0000x reference)
import functools

import jax
import jax.numpy as jnp
from jax.experimental import pallas as pl
from jax.experimental.pallas import tpu as pltpu

NUM_EMB = 1024
EMB_DIM = 256
COMMIT = 0.25
DECAY = 0.99
EPS = 1e-05

TILE_N = 1024


def _pass1_body(x_ref, e_ref, dist_ref, enc_ref, idx_ref, counts_ref, dw_ref):
    i = pl.program_id(0)
    x = x_ref[...]
    e = e_ref[...]
    cross = jax.lax.dot_general(
        x, e, (((0,), (1,)), ((), ())), preferred_element_type=jnp.float32)
    xn = jnp.sum(x * x, axis=0)
    en = jnp.sum(e * e, axis=1)
    dist = xn[:, None] + en[None, :] - 2.0 * cross
    dist_ref[...] = dist
    m = jnp.min(dist, axis=1, keepdims=True)
    iota_k = jax.lax.broadcasted_iota(jnp.int32, dist.shape, 1)
    idx = jnp.min(jnp.where(dist == m, iota_k, NUM_EMB), axis=1)
    idx_ref[...] = idx[:, None]
    enc = (iota_k == idx[:, None]).astype(jnp.float32)
    enc_ref[...] = enc
    part_counts = jnp.sum(enc, axis=0)[None, :]
    part_dw = jax.lax.dot_general(
        enc, x, (((0,), (1,)), ((), ())), preferred_element_type=jnp.float32)

    @pl.when(i == 0)
    def _init():
        counts_ref[...] = part_counts
        dw_ref[...] = part_dw

    @pl.when(i != 0)
    def _acc():
        counts_ref[...] += part_counts
        dw_ref[...] += part_dw


def _pass2_body(x_ref, idx_ref, counts_ref, dw_ref, ema_w_ref, ecs_ref,
                outq_ref, loss_ref, ppl_ref, nw_scratch, sse_scratch):
    i = pl.program_id(0)
    nsteps = pl.num_programs(0)
    n_total = jnp.float32(nsteps * TILE_N)

    @pl.when(i == 0)
    def _setup():
        counts = counts_ref[0, :]
        t = ecs_ref[0, :] * DECAY + (1.0 - DECAY) * counts
        n = jnp.sum(t)
        t = (t + EPS) / (n + NUM_EMB * EPS) * n
        nw = (ema_w_ref[...] * DECAY + (1.0 - DECAY) * dw_ref[...]) / t[:, None]
        nw_scratch[...] = nw
        p = counts / n_total
        ppl_ref[0, 0] = jnp.exp(-jnp.sum(p * jnp.log(p + 1e-10)))
        sse_scratch[0, 0] = 0.0

    x = x_ref[...]
    idx = idx_ref[...][:, 0]
    iota_k = jax.lax.broadcasted_iota(jnp.int32, (TILE_N, NUM_EMB), 1)
    enc = (iota_k == idx[:, None]).astype(jnp.float32)
    q = jax.lax.dot_general(
        nw_scratch[...], enc, (((0,), (1,)), ((), ())),
        preferred_element_type=jnp.float32)
    outq_ref[...] = q
    diff = q - x
    sse_scratch[0, 0] += jnp.sum(diff * diff)

    @pl.when(i == nsteps - 1)
    def _final():
        loss_ref[0, 0] = COMMIT * sse_scratch[0, 0] / (n_total * EMB_DIM)


def kernel(inputs, embedding_weight, ema_w, ema_cluster_size):
    D, B, T = inputs.shape
    N = B * T
    K = embedding_weight.shape[0]
    x2d = inputs.reshape(D, N)
    nt = N // TILE_N

    dist, enc, idx, counts, dw = pl.pallas_call(
        _pass1_body,
        grid=(nt,),
        in_specs=[
            pl.BlockSpec((D, TILE_N), lambda i: (0, i)),
            pl.BlockSpec((K, D), lambda i: (0, 0)),
        ],
        out_specs=[
            pl.BlockSpec((TILE_N, K), lambda i: (i, 0)),
            pl.BlockSpec((TILE_N, K), lambda i: (i, 0)),
            pl.BlockSpec((TILE_N, 1), lambda i: (i, 0)),
            pl.BlockSpec((1, K), lambda i: (0, 0)),
            pl.BlockSpec((K, D), lambda i: (0, 0)),
        ],
        out_shape=[
            jax.ShapeDtypeStruct((N, K), jnp.float32),
            jax.ShapeDtypeStruct((N, K), jnp.float32),
            jax.ShapeDtypeStruct((N, 1), jnp.int32),
            jax.ShapeDtypeStruct((1, K), jnp.float32),
            jax.ShapeDtypeStruct((K, D), jnp.float32),
        ],
    )(x2d, embedding_weight)

    outq, loss, ppl = pl.pallas_call(
        _pass2_body,
        grid=(nt,),
        in_specs=[
            pl.BlockSpec((D, TILE_N), lambda i: (0, i)),
            pl.BlockSpec((TILE_N, 1), lambda i: (i, 0)),
            pl.BlockSpec((1, K), lambda i: (0, 0)),
            pl.BlockSpec((K, D), lambda i: (0, 0)),
            pl.BlockSpec((K, D), lambda i: (0, 0)),
            pl.BlockSpec((1, K), lambda i: (0, 0)),
        ],
        out_specs=[
            pl.BlockSpec((D, TILE_N), lambda i: (0, i)),
            pl.BlockSpec(memory_space=pltpu.SMEM),
            pl.BlockSpec(memory_space=pltpu.SMEM),
        ],
        out_shape=[
            jax.ShapeDtypeStruct((D, N), jnp.float32),
            jax.ShapeDtypeStruct((1, 1), jnp.float32),
            jax.ShapeDtypeStruct((1, 1), jnp.float32),
        ],
        scratch_shapes=[
            pltpu.VMEM((K, D), jnp.float32),
            pltpu.SMEM((1, 1), jnp.float32),
        ],
    )(x2d, idx, counts, dw, ema_w, ema_cluster_size.reshape(1, K))

    return (loss[0, 0],
            outq.reshape(D, B, T),
            ppl[0, 0],
            enc.reshape(D, T, N * K // (D * T)),
            dist.reshape(D, T, N * K // (D * T)),
            idx)


# speedup vs baseline: 1.3963x; 1.3963x over previous
import jax
import jax.numpy as jnp
from jax.experimental import pallas as pl
from jax.experimental.pallas import tpu as pltpu

NUM_EMB = 1024
EMB_DIM = 256
COMMIT = 0.25
DECAY = 0.99
EPS = 1e-05

TILE_N = 2048


def _pass1_body(x_ref, e_ref, dist_ref, idx_ref, counts_ref, dw_ref):
    i = pl.program_id(0)
    x = x_ref[...]
    e = e_ref[...]
    cross = jax.lax.dot_general(
        x, e, (((0,), (1,)), ((), ())), preferred_element_type=jnp.float32)
    xn = jnp.sum(x * x, axis=0)
    en = jnp.sum(e * e, axis=1)
    dist = xn[:, None] + en[None, :] - 2.0 * cross
    dist_ref[...] = dist
    m = jnp.min(dist, axis=1, keepdims=True)
    iota_k = jax.lax.broadcasted_iota(jnp.int32, dist.shape, 1)
    idx = jnp.min(jnp.where(dist == m, iota_k, NUM_EMB), axis=1)
    idx_ref[...] = idx[:, None]
    enc = (iota_k == idx[:, None]).astype(jnp.float32)
    part_counts = jnp.sum(enc, axis=0)[None, :]
    part_dw = jax.lax.dot_general(
        enc, x, (((0,), (1,)), ((), ())), preferred_element_type=jnp.float32)

    @pl.when(i == 0)
    def _init():
        counts_ref[...] = part_counts
        dw_ref[...] = part_dw

    @pl.when(i != 0)
    def _acc():
        counts_ref[...] += part_counts
        dw_ref[...] += part_dw


def _pass2_body(x_ref, idx_ref, counts_ref, dw_ref, ema_w_ref, ecs_ref,
                outq_ref, loss_ref, ppl_ref, nw_scratch, sse_scratch):
    i = pl.program_id(0)
    nsteps = pl.num_programs(0)
    n_total = jnp.float32(nsteps * TILE_N)

    @pl.when(i == 0)
    def _setup():
        counts = counts_ref[0, :]
        t = ecs_ref[0, :] * DECAY + (1.0 - DECAY) * counts
        n = jnp.sum(t)
        t = (t + EPS) / (n + NUM_EMB * EPS) * n
        nw = (ema_w_ref[...] * DECAY + (1.0 - DECAY) * dw_ref[...]) / t[:, None]
        nw_scratch[...] = nw
        p = counts / n_total
        ppl_ref[0, 0] = jnp.exp(-jnp.sum(p * jnp.log(p + 1e-10)))
        sse_scratch[0, 0] = 0.0

    x = x_ref[...]
    idx = idx_ref[...][:, 0]
    iota_k = jax.lax.broadcasted_iota(jnp.int32, (TILE_N, NUM_EMB), 1)
    enc = (iota_k == idx[:, None]).astype(jnp.float32)
    q = jax.lax.dot_general(
        nw_scratch[...], enc, (((0,), (1,)), ((), ())),
        preferred_element_type=jnp.float32)
    outq_ref[...] = q
    diff = q - x
    sse_scratch[0, 0] += jnp.sum(diff * diff)

    @pl.when(i == nsteps - 1)
    def _final():
        loss_ref[0, 0] = COMMIT * sse_scratch[0, 0] / (n_total * EMB_DIM)


def kernel(inputs, embedding_weight, ema_w, ema_cluster_size):
    D, B, T = inputs.shape
    N = B * T
    K = embedding_weight.shape[0]
    nt = N // TILE_N
    L = N * K // (D * T)
    dpt = TILE_N * K // (T * L)
    x2d = inputs.reshape(D, N)

    dist, idx, counts, dw = pl.pallas_call(
        _pass1_body,
        grid=(nt,),
        in_specs=[
            pl.BlockSpec((D, TILE_N), lambda i: (0, i)),
            pl.BlockSpec((K, D), lambda i: (0, 0)),
        ],
        out_specs=[
            pl.BlockSpec((TILE_N, K), lambda i: (i, 0)),
            pl.BlockSpec((TILE_N, 1), lambda i: (i, 0)),
            pl.BlockSpec((1, K), lambda i: (0, 0)),
            pl.BlockSpec((K, D), lambda i: (0, 0)),
        ],
        out_shape=[
            jax.ShapeDtypeStruct((N, K), jnp.float32),
            jax.ShapeDtypeStruct((N, 1), jnp.int32),
            jax.ShapeDtypeStruct((1, K), jnp.float32),
            jax.ShapeDtypeStruct((K, D), jnp.float32),
        ],
    )(x2d, embedding_weight)

    outq, loss, ppl = pl.pallas_call(
        _pass2_body,
        grid=(nt,),
        in_specs=[
            pl.BlockSpec((D, TILE_N), lambda i: (0, i)),
            pl.BlockSpec((TILE_N, 1), lambda i: (i, 0)),
            pl.BlockSpec((1, K), lambda i: (0, 0)),
            pl.BlockSpec((K, D), lambda i: (0, 0)),
            pl.BlockSpec((K, D), lambda i: (0, 0)),
            pl.BlockSpec((1, K), lambda i: (0, 0)),
        ],
        out_specs=[
            pl.BlockSpec((D, TILE_N), lambda i: (0, i)),
            pl.BlockSpec(memory_space=pltpu.SMEM),
            pl.BlockSpec(memory_space=pltpu.SMEM),
        ],
        out_shape=[
            jax.ShapeDtypeStruct((D, N), jnp.float32),
            jax.ShapeDtypeStruct((1, 1), jnp.float32),
            jax.ShapeDtypeStruct((1, 1), jnp.float32),
        ],
        scratch_shapes=[
            pltpu.VMEM((K, D), jnp.float32),
            pltpu.SMEM((1, 1), jnp.float32),
        ],
    )(x2d, idx, counts, dw, ema_w, ema_cluster_size.reshape(1, K))

    idx1 = idx[:, 0]
    g = T // (N // D)
    rep = jnp.repeat(idx1.reshape(D, N // D), g, axis=1)
    kk = (jnp.arange(T, dtype=jnp.int32)[:, None] % g) * L + jnp.arange(L, dtype=jnp.int32)[None, :]
    enc3 = (rep[:, :, None] == kk[None, :, :]).astype(jnp.float32)
    return (loss[0, 0], outq.reshape(D, B, T), ppl[0, 0], enc3,
            dist.reshape(D, T, L), idx)


# speedup vs baseline: 1.5386x; 1.1018x over previous
import jax
import jax.numpy as jnp
from jax.experimental import pallas as pl
from jax.experimental.pallas import tpu as pltpu

NUM_EMB = 1024
EMB_DIM = 256
COMMIT = 0.25
DECAY = 0.99
EPS = 1e-05

TILE_N = 1024


def _pass1_body(x_ref, e_ref, dist_ref, idx_ref, counts_ref, dw_ref, xn2_ref):
    j = pl.program_id(0)
    b = j // 2
    h = j % 2
    x = x_ref[:, b, pl.ds(h * TILE_N, TILE_N)]
    e = e_ref[...]
    cross = jax.lax.dot_general(
        x, e, (((0,), (1,)), ((), ())), preferred_element_type=jnp.float32)
    xn = jnp.sum(x * x, axis=0)
    en = jnp.sum(e * e, axis=1)
    dist = xn[:, None] + en[None, :] - 2.0 * cross
    dist_ref[...] = dist
    m = jnp.min(dist, axis=1, keepdims=True)
    iota_k = jax.lax.broadcasted_iota(jnp.int32, dist.shape, 1)
    idx = jnp.min(jnp.where(dist == m, iota_k, NUM_EMB), axis=1)
    idx_ref[...] = idx[:, None]
    enc = (iota_k == idx[:, None]).astype(jnp.float32)
    part_counts = jnp.sum(enc, axis=0)[None, :]
    part_dw = jax.lax.dot_general(
        enc, x, (((0,), (1,)), ((), ())), preferred_element_type=jnp.float32)

    @pl.when(j == 0)
    def _init():
        counts_ref[...] = part_counts
        dw_ref[...] = part_dw
        xn2_ref[0, 0] = jnp.sum(xn)

    @pl.when(j != 0)
    def _acc():
        counts_ref[...] += part_counts
        dw_ref[...] += part_dw
        xn2_ref[0, 0] += jnp.sum(xn)


def _pass2_body(idx_ref, counts_ref, dw_ref, ema_w_ref, ecs_ref, xn2_ref,
                outq_ref, loss_ref, ppl_ref, nw_scratch):
    j = pl.program_id(0)
    nsteps = pl.num_programs(0)
    n_total = jnp.float32(nsteps * TILE_N)

    @pl.when(j == 0)
    def _setup():
        counts = counts_ref[0, :]
        t = ecs_ref[0, :] * DECAY + (1.0 - DECAY) * counts
        n = jnp.sum(t)
        t = (t + EPS) / (n + NUM_EMB * EPS) * n
        dwv = dw_ref[...]
        nw = (ema_w_ref[...] * DECAY + (1.0 - DECAY) * dwv) / t[:, None]
        nw_scratch[...] = nw
        p = counts / n_total
        ppl_ref[0, 0] = jnp.exp(-jnp.sum(p * jnp.log(p + 1e-10)))
        sse = (jnp.sum(jnp.sum(nw * nw, axis=1) * counts)
               - 2.0 * jnp.sum(nw * dwv) + xn2_ref[0, 0])
        loss_ref[0, 0] = COMMIT * sse / (n_total * EMB_DIM)

    b = j // 2
    h = j % 2
    idx = idx_ref[...][:, 0]
    iota_k = jax.lax.broadcasted_iota(jnp.int32, (TILE_N, NUM_EMB), 1)
    enc = (iota_k == idx[:, None]).astype(jnp.float32)
    q = jax.lax.dot_general(
        nw_scratch[...], enc, (((0,), (1,)), ((), ())),
        preferred_element_type=jnp.float32)
    outq_ref[:, b, pl.ds(h * TILE_N, TILE_N)] = q


def kernel(inputs, embedding_weight, ema_w, ema_cluster_size):
    D, B, T = inputs.shape
    N = B * T
    K = embedding_weight.shape[0]
    nt = N // TILE_N
    L = N * K // (D * T)

    dist, idx, counts, dw, xn2 = pl.pallas_call(
        _pass1_body,
        grid=(nt,),
        in_specs=[
            pl.BlockSpec((D, B, T), lambda j: (0, 0, 0)),
            pl.BlockSpec((K, D), lambda j: (0, 0)),
        ],
        out_specs=[
            pl.BlockSpec((TILE_N, K), lambda j: (j, 0)),
            pl.BlockSpec((TILE_N, 1), lambda j: (j, 0)),
            pl.BlockSpec((1, K), lambda j: (0, 0)),
            pl.BlockSpec((K, D), lambda j: (0, 0)),
            pl.BlockSpec(memory_space=pltpu.SMEM),
        ],
        out_shape=[
            jax.ShapeDtypeStruct((N, K), jnp.float32),
            jax.ShapeDtypeStruct((N, 1), jnp.int32),
            jax.ShapeDtypeStruct((1, K), jnp.float32),
            jax.ShapeDtypeStruct((K, D), jnp.float32),
            jax.ShapeDtypeStruct((1, 1), jnp.float32),
        ],
    )(inputs, embedding_weight)

    outq, loss, ppl = pl.pallas_call(
        _pass2_body,
        grid=(nt,),
        in_specs=[
            pl.BlockSpec((TILE_N, 1), lambda j: (j, 0)),
            pl.BlockSpec((1, K), lambda j: (0, 0)),
            pl.BlockSpec((K, D), lambda j: (0, 0)),
            pl.BlockSpec((K, D), lambda j: (0, 0)),
            pl.BlockSpec((1, K), lambda j: (0, 0)),
            pl.BlockSpec(memory_space=pltpu.SMEM),
        ],
        out_specs=[
            pl.BlockSpec((D, B, T), lambda j: (0, 0, 0)),
            pl.BlockSpec(memory_space=pltpu.SMEM),
            pl.BlockSpec(memory_space=pltpu.SMEM),
        ],
        out_shape=[
            jax.ShapeDtypeStruct((D, B, T), jnp.float32),
            jax.ShapeDtypeStruct((1, 1), jnp.float32),
            jax.ShapeDtypeStruct((1, 1), jnp.float32),
        ],
        scratch_shapes=[
            pltpu.VMEM((K, D), jnp.float32),
        ],
    )(idx, counts, dw, ema_w, ema_cluster_size.reshape(1, K), xn2)

    idx1 = idx[:, 0]
    g = T // (N // D)
    rep = jnp.repeat(idx1.reshape(D, N // D), g, axis=1)
    kk = (jnp.arange(T, dtype=jnp.int32)[:, None] % g) * L + jnp.arange(L, dtype=jnp.int32)[None, :]
    enc3 = (rep[:, :, None] == kk[None, :, :]).astype(jnp.float32)
    return (loss[0, 0], outq, ppl[0, 0], enc3,
            dist.reshape(D, T, L), idx)
